# baseline (speedup 1.0000x reference)
import functools

import jax
import jax.numpy as jnp
from jax import lax
from jax.experimental import pallas as pl
from jax.experimental.pallas import tpu as pltpu
from jax.experimental.pallas import tpu_sc as plsc

_INPUT_SIZE = 26
_BATCH = 16384
_DIM = 32
_TABLE = 1 << 20
_MASK = _TABLE - 1
_TILES = _TABLE // 8
_NW = 32
_BPW = _BATCH // _NW
_L = 16
_GATHER = 32
_NGATHER = _BPW // _GATHER
_TCB = 2048
_RMAGIC = 12582912.0

_mesh = plsc.VectorSubcoreMesh(core_axis_name="c", subcore_axis_name="s")


def _hash_tc(x_ref, d_ref, h0_ref, lin_ref, sub_ref):
    bits = x_ref[...] > 0.5
    contrib = jnp.where(bits, d_ref[...], 0)
    h = h0_ref[0, 0] + jnp.sum(contrib, axis=1, dtype=jnp.int32)
    idx = jnp.bitwise_and(h, _MASK).reshape(_TCB // 128, 128)
    lin_ref[...] = idx
    sub_ref[...] = idx


_hash_idx = pl.pallas_call(
    _hash_tc,
    grid=(_BATCH // _TCB,),
    in_specs=[
        pl.BlockSpec((_TCB, _INPUT_SIZE), lambda i: (i, 0)),
        pl.BlockSpec((1, _INPUT_SIZE), lambda i: (0, 0)),
        pl.BlockSpec((1, 1), lambda i: (0, 0)),
    ],
    out_specs=[
        pl.BlockSpec((_TCB // 128, 128), lambda i: (i, 0)),
        pl.BlockSpec((_TCB // 128, 128), lambda i: (i, 0)),
    ],
    out_shape=[
        jax.ShapeDtypeStruct((_BATCH // 128, 128), jnp.int32),
        jax.ShapeDtypeStruct((_BATCH // 128, 128), jnp.int32),
    ],
)


@functools.partial(
    pl.kernel,
    mesh=_mesh,
    out_type=jax.ShapeDtypeStruct((_BATCH, _DIM), jnp.float32),
    scratch_types=[
        pltpu.VMEM((_BPW // 128, 128), jnp.int32),
        pltpu.VMEM((_BPW // 128, 128), jnp.int32),
        pltpu.VMEM((2, _GATHER, _DIM), jnp.float32),
        pltpu.VMEM((2, _GATHER, _DIM), jnp.float32),
        pltpu.SemaphoreType.DMA,
        pltpu.SemaphoreType.DMA,
    ],
    compiler_params=pltpu.CompilerParams(use_tc_tiling_on_sc=True),
)
def _gather_quant(lin_hbm, sub_hbm, feat_hbm, out_hbm,
                  lin_v, sub_v, tiles_v, rows_v, sem, osem):
    wid = lax.axis_index("s") * 2 + lax.axis_index("c")
    base = wid * _BPW
    nidx = _BPW // 128

    pltpu.sync_copy(lin_hbm.at[pl.ds(wid * nidx, nidx)], lin_v)
    pltpu.sync_copy(sub_hbm.at[pl.ds(wid * nidx, nidx)], sub_v)

    def _start(g):
        r, off = divmod(g * _GATHER, 128)
        sv = lin_v[r, pl.ds(off, _GATHER)]
        cps = []
        for j in range(_GATHER):
            cps.append(pltpu.async_copy(
                feat_hbm.at[sv[j]], tiles_v.at[g % 2, j], sem))
        return cps

    cp = _start(0)
    out_cp = None
    for g in range(_NGATHER):
        nxt = _start(g + 1) if g + 1 < _NGATHER else None
        for c in cp:
            c.wait()

        def _quant(b, carry, g=g):
            for h in range(_DIM // _L):
                v = tiles_v[g % 2, b, pl.ds(h * _L, _L)]
                v = jnp.minimum(jnp.maximum(v, -1.0), 127.0 / 128.0)
                y = v * 128.0
                q = (y + _RMAGIC) - _RMAGIC
                rows_v[g % 2, b, pl.ds(h * _L, _L)] = q * (1.0 / 128.0)
            return carry

        lax.fori_loop(0, _GATHER, _quant, 0)
        if out_cp is not None:
            out_cp.wait()
        out_cp = pltpu.async_copy(
            rows_v.at[g % 2],
            out_hbm.at[pl.ds(base + g * _GATHER, _GATHER)], osem)
        cp = nxt
    out_cp.wait()


def kernel(x, features, hashs):
    hi = hashs.astype(jnp.int32)
    dv = (hi[:, 1] - hi[:, 0]).reshape(1, _INPUT_SIZE)
    h0 = jnp.sum(hi[:, 0], dtype=jnp.int32).reshape(1, 1)
    lin, sub = _hash_idx(x, dv, h0)
    return _gather_quant(lin, sub, features)


# speedup vs baseline: 1.0107x; 1.0107x over previous
import functools

import jax
import jax.numpy as jnp
from jax import lax
from jax.experimental import pallas as pl
from jax.experimental.pallas import tpu as pltpu
from jax.experimental.pallas import tpu_sc as plsc

_INPUT_SIZE = 26
_BATCH = 16384
_DIM = 32
_TABLE = 1 << 20
_MASK = _TABLE - 1
_NW = 32
_BPW = _BATCH // _NW
_L = 16
_GATHER = 32
_NGATHER = _BPW // _GATHER
_DEPTH = 4
_TCB = 2048
_RMAGIC = 12582912.0

_mesh = plsc.VectorSubcoreMesh(core_axis_name="c", subcore_axis_name="s")


def _hash_tc(x_ref, d_ref, h0_ref, lin_ref):
    bits = x_ref[...] > 0.5
    contrib = jnp.where(bits, d_ref[...], 0)
    h = h0_ref[0, 0] + jnp.sum(contrib, axis=1, dtype=jnp.int32)
    lin_ref[...] = jnp.bitwise_and(h, _MASK).reshape(_TCB // 128, 128)


_hash_idx = pl.pallas_call(
    _hash_tc,
    grid=(_BATCH // _TCB,),
    in_specs=[
        pl.BlockSpec((_TCB, _INPUT_SIZE), lambda i: (i, 0)),
        pl.BlockSpec((1, _INPUT_SIZE), lambda i: (0, 0)),
        pl.BlockSpec((1, 1), lambda i: (0, 0)),
    ],
    out_specs=pl.BlockSpec((_TCB // 128, 128), lambda i: (i, 0)),
    out_shape=jax.ShapeDtypeStruct((_BATCH // 128, 128), jnp.int32),
)


@functools.partial(
    pl.kernel,
    mesh=_mesh,
    out_type=jax.ShapeDtypeStruct((_BATCH, _DIM), jnp.float32),
    scratch_types=[
        pltpu.VMEM((_BPW // 128, 128), jnp.int32),
        pltpu.VMEM((_DEPTH, _GATHER, _DIM), jnp.float32),
        pltpu.VMEM((2, _GATHER, _DIM), jnp.float32),
        pltpu.SemaphoreType.DMA,
        pltpu.SemaphoreType.DMA,
        pltpu.SemaphoreType.DMA,
        pltpu.SemaphoreType.DMA,
        pltpu.SemaphoreType.DMA,
    ],
    compiler_params=pltpu.CompilerParams(use_tc_tiling_on_sc=True),
)
def _gather_quant(lin_hbm, feat_hbm, out_hbm,
                  lin_v, tiles_v, rows_v, s0, s1, s2, s3, osem):
    wid = lax.axis_index("s") * 2 + lax.axis_index("c")
    base = wid * _BPW
    nidx = _BPW // 128
    sems = [s0, s1, s2, s3]

    pltpu.sync_copy(lin_hbm.at[pl.ds(wid * nidx, nidx)], lin_v)

    def _start(g):
        r, off = divmod(g * _GATHER, 128)
        sv = lin_v[r, pl.ds(off, _GATHER)]
        buf = g % _DEPTH
        cps = []
        for j in range(_GATHER):
            cps.append(pltpu.async_copy(
                feat_hbm.at[sv[j]], tiles_v.at[buf, j], sems[buf]))
        return cps

    cps = [_start(g) for g in range(_DEPTH)]
    out_cp = None
    for g in range(_NGATHER):
        for c in cps[g % _DEPTH]:
            c.wait()

        def _quant(b, carry, g=g):
            for h in range(_DIM // _L):
                v = tiles_v[g % _DEPTH, b, pl.ds(h * _L, _L)]
                v = jnp.minimum(jnp.maximum(v, -1.0), 127.0 / 128.0)
                y = v * 128.0
                q = (y + _RMAGIC) - _RMAGIC
                rows_v[g % 2, b, pl.ds(h * _L, _L)] = q * (1.0 / 128.0)
            return carry

        lax.fori_loop(0, _GATHER, _quant, 0)
        if out_cp is not None:
            out_cp.wait()
        out_cp = pltpu.async_copy(
            rows_v.at[g % 2],
            out_hbm.at[pl.ds(base + g * _GATHER, _GATHER)], osem)
        if g + _DEPTH < _NGATHER:
            cps[g % _DEPTH] = _start(g + _DEPTH)
    out_cp.wait()


def kernel(x, features, hashs):
    hi = hashs.astype(jnp.int32)
    dv = (hi[:, 1] - hi[:, 0]).reshape(1, _INPUT_SIZE)
    h0 = jnp.sum(hi[:, 0], dtype=jnp.int32).reshape(1, 1)
    lin = _hash_idx(x, dv, h0)
    return _gather_quant(lin, features)


# speedup vs baseline: 1.0147x; 1.0040x over previous
import functools

import jax
import jax.numpy as jnp
from jax import lax
from jax.experimental import pallas as pl
from jax.experimental.pallas import tpu as pltpu
from jax.experimental.pallas import tpu_sc as plsc

_INPUT_SIZE = 26
_BATCH = 16384
_DIM = 32
_TABLE = 1 << 20
_MASK = _TABLE - 1
_NW = 32
_BPW = _BATCH // _NW
_L = 16
_GATHER = 128
_NGATHER = _BPW // _GATHER
_DEPTH = 4
_TCB = 2048
_RMAGIC = 12582912.0

_mesh = plsc.VectorSubcoreMesh(core_axis_name="c", subcore_axis_name="s")


def _hash_tc(x_ref, d_ref, h0_ref, lin_ref):
    bits = x_ref[...] > 0.5
    contrib = jnp.where(bits, d_ref[...], 0)
    h = h0_ref[0, 0] + jnp.sum(contrib, axis=1, dtype=jnp.int32)
    lin_ref[...] = jnp.bitwise_and(h, _MASK).reshape(_TCB // 128, 128)


_hash_idx = pl.pallas_call(
    _hash_tc,
    grid=(_BATCH // _TCB,),
    in_specs=[
        pl.BlockSpec((_TCB, _INPUT_SIZE), lambda i: (i, 0)),
        pl.BlockSpec((1, _INPUT_SIZE), lambda i: (0, 0)),
        pl.BlockSpec((1, 1), lambda i: (0, 0)),
    ],
    out_specs=pl.BlockSpec((_TCB // 128, 128), lambda i: (i, 0)),
    out_shape=jax.ShapeDtypeStruct((_BATCH // 128, 128), jnp.int32),
)


@functools.partial(
    pl.kernel,
    mesh=_mesh,
    out_type=jax.ShapeDtypeStruct((_BATCH, _DIM), jnp.float32),
    scratch_types=[
        pltpu.VMEM((_BPW // 128, 128), jnp.int32),
        pltpu.VMEM((_DEPTH, _GATHER, _DIM), jnp.float32),
        pltpu.VMEM((2, _GATHER, _DIM), jnp.float32),
        pltpu.SemaphoreType.DMA,
        pltpu.SemaphoreType.DMA,
        pltpu.SemaphoreType.DMA,
        pltpu.SemaphoreType.DMA,
        pltpu.SemaphoreType.DMA,
    ],
    compiler_params=pltpu.CompilerParams(use_tc_tiling_on_sc=True),
)
def _gather_quant(lin_hbm, feat_hbm, out_hbm,
                  lin_v, tiles_v, rows_v, s0, s1, s2, s3, osem):
    wid = lax.axis_index("s") * 2 + lax.axis_index("c")
    base = wid * _BPW
    nidx = _BPW // 128
    sems = [s0, s1, s2, s3]

    pltpu.sync_copy(lin_hbm.at[pl.ds(wid * nidx, nidx)], lin_v)

    def _start(g):
        r, off = divmod(g * _GATHER, 128)
        sv = lin_v[r, pl.ds(off, _GATHER)]
        buf = g % _DEPTH
        cps = []
        for j in range(_GATHER):
            cps.append(pltpu.async_copy(
                feat_hbm.at[sv[j]], tiles_v.at[buf, j], sems[buf]))
        return cps

    cps = [_start(g) for g in range(_DEPTH)]
    out_cp = None
    for g in range(_NGATHER):
        for c in cps[g % _DEPTH]:
            c.wait()

        def _quant(b, carry, g=g):
            for h in range(_DIM // _L):
                v = tiles_v[g % _DEPTH, b, pl.ds(h * _L, _L)]
                v = jnp.minimum(jnp.maximum(v, -1.0), 127.0 / 128.0)
                y = v * 128.0
                q = (y + _RMAGIC) - _RMAGIC
                rows_v[g % 2, b, pl.ds(h * _L, _L)] = q * (1.0 / 128.0)
            return carry

        lax.fori_loop(0, _GATHER, _quant, 0)
        if out_cp is not None:
            out_cp.wait()
        out_cp = pltpu.async_copy(
            rows_v.at[g % 2],
            out_hbm.at[pl.ds(base + g * _GATHER, _GATHER)], osem)
        if g + _DEPTH < _NGATHER:
            cps[g % _DEPTH] = _start(g + _DEPTH)
    out_cp.wait()


def kernel(x, features, hashs):
    hi = hashs.astype(jnp.int32)
    dv = (hi[:, 1] - hi[:, 0]).reshape(1, _INPUT_SIZE)
    h0 = jnp.sum(hi[:, 0], dtype=jnp.int32).reshape(1, 1)
    lin = _hash_idx(x, dv, h0)
    return _gather_quant(lin, features)
